# baseline (speedup 1.0000x reference)
import functools

import jax
import jax.numpy as jnp
from jax import lax
from jax.experimental import pallas as pl
from jax.experimental.pallas import tpu as pltpu
from jax.experimental.pallas import tpu_sc as plsc

_NC = 2
_NS = 16
_NW = _NC * _NS
_LANES = 16


def _sc_gather_max(table, idx_flat, C, K, VPC):
    R = idx_flat.shape[0] // K
    IPC = VPC * K
    NBUF = 2

    PPC = R // (VPC * _NS)
    CH0 = (int(PPC * 0.75) // NBUF) * NBUF
    CH1 = PPC - CH0
    assert CH1 >= NBUF and CH0 % NBUF == 0 and CH1 % NBUF == 0, (PPC, CH0, CH1)
    idx_flat = jnp.pad(idx_flat, (0, (CH0 - CH1) * IPC))

    mesh = plsc.VectorSubcoreMesh(core_axis_name="c", subcore_axis_name="s")

    @functools.partial(
        pl.kernel,
        mesh=mesh,
        out_type=jax.ShapeDtypeStruct((R, C), jnp.float32),
        scratch_types=[
            pltpu.VMEM((CH0 * IPC,), jnp.int32),
        ] + [pltpu.VMEM((IPC, C), jnp.float32)] * NBUF
          + [pltpu.VMEM((VPC, C), jnp.float32)] * NBUF
          + [pltpu.SemaphoreType.DMA] * NBUF
          + [pltpu.SemaphoreType.DMA] * NBUF,
    )
    def gk(table_hbm, idx_hbm, out_hbm, idx_v, *bufs):
        rows = bufs[:NBUF]
        outs = bufs[NBUF:2 * NBUF]
        gsem = bufs[2 * NBUF:3 * NBUF]
        osem = bufs[3 * NBUF:4 * NBUF]
        sid = lax.axis_index("s")
        cid = lax.axis_index("c")
        NCHUNK = jnp.where(cid == 1, CH0, CH1)
        base = jnp.where(cid == 1, sid * CH0, _NS * CH0 + sid * CH1) * VPC
        pltpu.sync_copy(idx_hbm.at[pl.ds(base * K, CH0 * IPC)], idx_v)

        def fire(ci, b):
            pltpu.async_copy(
                table_hbm.at[idx_v.at[pl.ds(ci * IPC, IPC)]], rows[b], gsem[b])

        def out_slice(ci):
            return out_hbm.at[pl.ds(base + ci * VPC, VPC)]

        for b in range(NBUF):
            fire(b, b)

        def round_body(q, carry):
            for b in range(NBUF):
                ci = q * NBUF + b
                pltpu.make_async_copy(
                    table_hbm.at[idx_v.at[pl.ds(0, IPC)]], rows[b], gsem[b]
                ).wait()
                @pl.when(q > 0)
                def _(b=b, ci=ci):
                    pltpu.make_async_copy(
                        outs[b], out_slice(ci - NBUF), osem[b]).wait()
                for v in range(VPC):
                    def lane_body(c16, carry2, v=v, b=b):
                        off = c16 * _LANES
                        acc = rows[b][v * K, pl.ds(off, _LANES)]
                        for j in range(1, K):
                            acc = jnp.maximum(
                                acc, rows[b][v * K + j, pl.ds(off, _LANES)])
                        outs[b][v, pl.ds(off, _LANES)] = acc
                        return carry2
                    lax.fori_loop(0, C // _LANES, lane_body, 0)
                pltpu.async_copy(outs[b], out_slice(ci), osem[b])

                @pl.when(ci + NBUF < NCHUNK)
                def _(b=b, ci=ci):
                    fire(ci + NBUF, b)
            return carry
        lax.fori_loop(0, NCHUNK // NBUF, round_body, 0)

        for b in range(NBUF):
            pltpu.make_async_copy(
                outs[b], out_slice(NCHUNK - NBUF + b), osem[b]).wait()

    return gk(table, idx_flat)


def _tc_proj(x, Wn, Wd, b2d, BR):
    R, Cin = x.shape
    Ca = Wn.shape[1]
    Cc = Wd.shape[1]

    def body(x_ref, wn_ref, wd_ref, b_ref, a_ref, c_ref):
        xv = x_ref[...]
        a_ref[...] = jnp.dot(xv, wn_ref[...], preferred_element_type=jnp.float32, precision=lax.Precision.HIGHEST)
        c_ref[...] = jnp.dot(xv, wd_ref[...], preferred_element_type=jnp.float32, precision=lax.Precision.HIGHEST) + b_ref[...]

    return pl.pallas_call(
        body,
        grid=(R // BR,),
        in_specs=[
            pl.BlockSpec((BR, Cin), lambda i: (i, 0)),
            pl.BlockSpec((Cin, Ca), lambda i: (0, 0)),
            pl.BlockSpec((Cin, Cc), lambda i: (0, 0)),
            pl.BlockSpec((1, Cc), lambda i: (0, 0)),
        ],
        out_specs=[
            pl.BlockSpec((BR, Ca), lambda i: (i, 0)),
            pl.BlockSpec((BR, Cc), lambda i: (i, 0)),
        ],
        out_shape=[jax.ShapeDtypeStruct((R, Ca), jnp.float32),
                   jax.ShapeDtypeStruct((R, Cc), jnp.float32)],
    )(x, Wn, Wd, b2d)


def _tc_act_proj(m, c, Wn, Wd, b2d, BR):
    R, Cm = m.shape
    Cin = c.shape[1]
    Ca = Wn.shape[1]
    Cc = Wd.shape[1]

    def body(m_ref, c_ref, wn_ref, wd_ref, b_ref, x_ref, a_ref, cn_ref):
        t = m_ref[...][:, :Cin] + c_ref[...]
        xv = jnp.maximum(t, 0.2 * t)
        x_ref[...] = xv
        a_ref[...] = jnp.dot(xv, wn_ref[...], preferred_element_type=jnp.float32, precision=lax.Precision.HIGHEST)
        cn_ref[...] = jnp.dot(xv, wd_ref[...], preferred_element_type=jnp.float32, precision=lax.Precision.HIGHEST) + b_ref[...]

    return pl.pallas_call(
        body,
        grid=(R // BR,),
        in_specs=[
            pl.BlockSpec((BR, Cm), lambda i: (i, 0)),
            pl.BlockSpec((BR, Cin), lambda i: (i, 0)),
            pl.BlockSpec((Cin, Ca), lambda i: (0, 0)),
            pl.BlockSpec((Cin, Cc), lambda i: (0, 0)),
            pl.BlockSpec((1, Cc), lambda i: (0, 0)),
        ],
        out_specs=[
            pl.BlockSpec((BR, Cin), lambda i: (i, 0)),
            pl.BlockSpec((BR, Ca), lambda i: (i, 0)),
            pl.BlockSpec((BR, Cc), lambda i: (i, 0)),
        ],
        out_shape=[
            jax.ShapeDtypeStruct((R, Cin), jnp.float32),
            jax.ShapeDtypeStruct((R, Ca), jnp.float32),
            jax.ShapeDtypeStruct((R, Cc), jnp.float32),
        ],
    )(m, c, Wn, Wd, b2d)


def _tc_pool_proj(W, x0, x1, x2, m3, c3, Wn, Wd, b2d, NCH):
    B, M, N = W.shape
    Nc = N // NCH
    Gw = x0.shape[2] + x1.shape[2] + x2.shape[2] + m3.shape[2]
    Cout = Wn.shape[1]

    def body(w_ref, x0_ref, x1_ref, x2_ref, m3_ref, c3_ref, wn_ref, wd_ref,
             b_ref, a_ref, cn_ref, acc, wsum):
        nc = pl.program_id(1)
        w = w_ref[0]
        t = m3_ref[0] + c3_ref[0]
        x3 = jnp.maximum(t, 0.2 * t)
        g = jnp.concatenate([x0_ref[0], x1_ref[0], x2_ref[0], x3], axis=-1)
        pacc = jnp.dot(w, g, preferred_element_type=jnp.float32, precision=lax.Precision.HIGHEST)
        pw = jnp.sum(w, axis=1).reshape(M, 1)

        @pl.when(nc == 0)
        def _():
            acc[...] = pacc
            wsum[...] = pw

        @pl.when(nc > 0)
        def _():
            acc[...] += pacc
            wsum[...] += pw

        @pl.when(nc == NCH - 1)
        def _():
            vj = acc[...] / (wsum[...] + 1e-5)
            a_ref[0] = jnp.dot(vj, wn_ref[...], preferred_element_type=jnp.float32, precision=lax.Precision.HIGHEST)
            cn_ref[0] = jnp.dot(vj, wd_ref[...], preferred_element_type=jnp.float32, precision=lax.Precision.HIGHEST) + b_ref[...]

    return pl.pallas_call(
        body,
        grid=(B, NCH),
        in_specs=[
            pl.BlockSpec((1, M, Nc), lambda b, n: (b, 0, n)),
            pl.BlockSpec((1, Nc, x0.shape[2]), lambda b, n: (b, n, 0)),
            pl.BlockSpec((1, Nc, x1.shape[2]), lambda b, n: (b, n, 0)),
            pl.BlockSpec((1, Nc, x2.shape[2]), lambda b, n: (b, n, 0)),
            pl.BlockSpec((1, Nc, m3.shape[2]), lambda b, n: (b, n, 0)),
            pl.BlockSpec((1, Nc, c3.shape[2]), lambda b, n: (b, n, 0)),
            pl.BlockSpec((Gw, Cout), lambda b, n: (0, 0)),
            pl.BlockSpec((Gw, Cout), lambda b, n: (0, 0)),
            pl.BlockSpec((1, Cout), lambda b, n: (0, 0)),
        ],
        out_specs=[
            pl.BlockSpec((1, M, Cout), lambda b, n: (b, 0, 0)),
            pl.BlockSpec((1, M, Cout), lambda b, n: (b, 0, 0)),
        ],
        out_shape=[jax.ShapeDtypeStruct((B, M, Cout), jnp.float32)] * 2,
        scratch_shapes=[
            pltpu.VMEM((M, Gw), jnp.float32),
            pltpu.VMEM((M, 1), jnp.float32),
        ],
    )(W, x0, x1, x2, m3, c3, Wn, Wd, b2d)


def _tc_head(m3, c3, s1, s2, W1, b1, W2, b2, W3p, b3p):
    R = m3.shape[0]
    H1 = W1.shape[1]
    H2 = W2.shape[1]
    CO = W3p.shape[1]

    def body(m_ref, c_ref, s1_ref, s2_ref, w1_ref, b1_ref, w2_ref, b2_ref,
             w3_ref, b3_ref, o_ref):
        t = m_ref[...][:, :c_ref.shape[1]] + c_ref[...]
        s3 = jnp.maximum(t, 0.2 * t)
        j = jnp.concatenate([s1_ref[...], s2_ref[...], s3], axis=-1)
        h = jnp.dot(j, w1_ref[...], preferred_element_type=jnp.float32, precision=lax.Precision.HIGHEST) + b1_ref[...]
        h = jnp.maximum(h, 0.2 * h)
        h = jnp.dot(h, w2_ref[...], preferred_element_type=jnp.float32, precision=lax.Precision.HIGHEST) + b2_ref[...]
        h = jnp.maximum(h, 0.2 * h)
        o_ref[...] = jnp.dot(h, w3_ref[...], preferred_element_type=jnp.float32, precision=lax.Precision.HIGHEST) + b3_ref[...]

    return pl.pallas_call(
        body,
        out_shape=jax.ShapeDtypeStruct((R, CO), jnp.float32),
    )(m3, c3, s1, s2, W1, b1.reshape(1, H1), W2, b2.reshape(1, H2), W3p, b3p)


def _split_w(Wc, bc):
    Cin = Wc.shape[0] // 2
    return Wc[:Cin], Wc[Cin:] - Wc[:Cin], bc.reshape(1, -1)


def _flat_idx(idx, stride, R_pad):
    B, Rb, K = idx.shape
    idx = jnp.pad(idx.astype(jnp.int32), ((0, 0), (0, stride - Rb), (0, 0)))
    offs = (jnp.arange(B, dtype=jnp.int32) * stride)[:, None, None]
    return (idx + offs).reshape(R_pad * K)


def kernel(V, W, facesOneRingIdx, skeletonOneRingIdx,
           Wg1, bg1, Wg2, bg2, Wg3, bg3,
           Ws1, bs1, Ws2, bs2, Ws3, bs3,
           Wm1, bm1, Wm2, bm2, Wm3, bm3):
    B, N, _ = V.shape
    M = W.shape[1]
    K = facesOneRingIdx.shape[-1]

    NCH = 5
    N_pad = -(-N // (NCH * 128)) * (NCH * 128)
    R_pad = B * N_pad
    BR = 512
    RS = B * M

    Wn1, Wd1, b1 = _split_w(Wg1, bg1)
    Wn2, Wd2, b2 = _split_w(Wg2, bg2)
    Wn3, Wd3, b3 = _split_w(Wg3, bg3)
    Sn1, Sd1, sb1 = _split_w(Ws1, bs1)
    Sn2, Sd2, sb2 = _split_w(Ws2, bs2)
    Sn3, Sd3, sb3 = _split_w(Ws3, bs3)

    V_pad = jnp.pad(V, ((0, 0), (0, N_pad - N), (0, 0)))
    x0f = jnp.pad(V_pad.reshape(R_pad, 3), ((0, 0), (0, 5)))
    W_pad = jnp.pad(W, ((0, 0), (0, 0), (0, N_pad - N)))
    Wn1p = jnp.pad(Wn1, ((0, 5), (0, 64)))
    Wd1p = jnp.pad(Wd1, ((0, 5), (0, 0)))
    Sn3p = jnp.pad(Sn3, ((0, 0), (0, 64)))

    fidx = _flat_idx(facesOneRingIdx, N_pad, R_pad)
    sidx = _flat_idx(skeletonOneRingIdx, M, RS)

    a1, c1 = _tc_proj(x0f, Wn1p, Wd1p, b1, BR)
    m1 = _sc_gather_max(a1, fidx, 128, K, 4)
    x1, a2, c2 = _tc_act_proj(m1, c1, Wn2, Wd2, b2, BR)
    m2 = _sc_gather_max(a2, fidx, 128, K, 4)
    x2, a3, c3 = _tc_act_proj(m2, c2, Wn3, Wd3, b3, BR)
    m3 = _sc_gather_max(a3, fidx, 256, K, 4)

    def unflat(x, Cc):
        return x.reshape(B, N_pad, Cc)
    as1, cs1 = _tc_pool_proj(
        W_pad, V_pad, unflat(x1, 64), unflat(x2, 128), unflat(m3, 256),
        unflat(c3, 256), Sn1, Sd1, sb1, NCH=NCH)
    as1 = as1.reshape(RS, 256)
    cs1 = cs1.reshape(RS, 256)

    ms1 = _sc_gather_max(as1, sidx, 256, K, 4)
    s1, as2, cs2 = _tc_act_proj(ms1, cs1, Sn2, Sd2, sb2, RS)
    ms2 = _sc_gather_max(as2, sidx, 128, K, 4)
    s2, as3, cs3 = _tc_act_proj(ms2, cs2, Sn3p, Sd3, sb3, RS)
    ms3 = _sc_gather_max(as3, sidx, 128, K, 4)

    W3p = jnp.pad(Wm3, ((0, 0), (0, 125)))
    b3p = jnp.pad(bm3, (0, 125)).reshape(1, 128)
    out = _tc_head(ms3, cs3, s1, s2, Wm1, bm1, Wm2, bm2, W3p, b3p)
    return out[:, :3].reshape(B, M, 3)


# speedup vs baseline: 1.1120x; 1.1120x over previous
import functools

import jax
import jax.numpy as jnp
from jax import lax
from jax.experimental import pallas as pl
from jax.experimental.pallas import tpu as pltpu
from jax.experimental.pallas import tpu_sc as plsc

_NC = 2
_NS = 16
_NW = _NC * _NS
_LANES = 16


def _sc_gather_max(table, idx_flat, C, K, VPC):
    R = idx_flat.shape[0] // K
    IPC = VPC * K
    NBUF = 2

    PPC = R // (VPC * _NS)
    CH0 = (int(PPC * 0.75) // NBUF) * NBUF
    CH1 = PPC - CH0
    assert CH1 >= NBUF and CH0 % NBUF == 0 and CH1 % NBUF == 0, (PPC, CH0, CH1)
    idx_flat = jnp.pad(idx_flat, (0, (CH0 - CH1) * IPC))

    mesh = plsc.VectorSubcoreMesh(core_axis_name="c", subcore_axis_name="s")

    @functools.partial(
        pl.kernel,
        mesh=mesh,
        out_type=jax.ShapeDtypeStruct((R, C), jnp.float32),
        scratch_types=[
            pltpu.VMEM((CH0 * IPC,), jnp.int32),
        ] + [pltpu.VMEM((IPC, C), jnp.float32)] * NBUF
          + [pltpu.VMEM((VPC, C), jnp.float32)] * NBUF
          + [pltpu.SemaphoreType.DMA] * NBUF
          + [pltpu.SemaphoreType.DMA] * NBUF,
    )
    def gk(table_hbm, idx_hbm, out_hbm, idx_v, *bufs):
        rows = bufs[:NBUF]
        outs = bufs[NBUF:2 * NBUF]
        gsem = bufs[2 * NBUF:3 * NBUF]
        osem = bufs[3 * NBUF:4 * NBUF]
        sid = lax.axis_index("s")
        cid = lax.axis_index("c")

        def emit_pipeline(NCHUNK, base):
            pltpu.sync_copy(idx_hbm.at[pl.ds(base * K, CH0 * IPC)], idx_v)

            def fire(ci, b):
                pltpu.async_copy(
                    table_hbm.at[idx_v.at[pl.ds(ci * IPC, IPC)]],
                    rows[b], gsem[b])

            def out_slice(ci):
                return out_hbm.at[pl.ds(base + ci * VPC, VPC)]

            for b in range(NBUF):
                fire(b, b)

            def round_body(q, carry):
                for b in range(NBUF):
                    ci = q * NBUF + b
                    pltpu.make_async_copy(
                        table_hbm.at[idx_v.at[pl.ds(0, IPC)]], rows[b], gsem[b]
                    ).wait()
                    @pl.when(q > 0)
                    def _(b=b, ci=ci):
                        pltpu.make_async_copy(
                            outs[b], out_slice(ci - NBUF), osem[b]).wait()
                    for v in range(VPC):
                        def lane_body(c16, carry2, v=v, b=b):
                            off = c16 * _LANES
                            acc = rows[b][v * K, pl.ds(off, _LANES)]
                            for j in range(1, K):
                                acc = jnp.maximum(
                                    acc, rows[b][v * K + j, pl.ds(off, _LANES)])
                            outs[b][v, pl.ds(off, _LANES)] = acc
                            return carry2
                        lax.fori_loop(0, C // _LANES, lane_body, 0)
                    pltpu.async_copy(outs[b], out_slice(ci), osem[b])

                    @pl.when(ci + NBUF < NCHUNK)
                    def _(b=b, ci=ci):
                        fire(ci + NBUF, b)
                return carry
            lax.fori_loop(0, NCHUNK // NBUF, round_body, 0)

            for b in range(NBUF):
                pltpu.make_async_copy(
                    outs[b], out_slice(NCHUNK - NBUF + b), osem[b]).wait()

        @pl.when(cid == 0)
        def _():
            emit_pipeline(CH0, sid * CH0 * VPC)

        @pl.when(cid == 1)
        def _():
            emit_pipeline(CH1, (_NS * CH0 + sid * CH1) * VPC)

    return gk(table, idx_flat)


def _tc_proj(x, Wn, Wd, b2d, BR):
    R, Cin = x.shape
    Ca = Wn.shape[1]
    Cc = Wd.shape[1]

    def body(x_ref, wn_ref, wd_ref, b_ref, a_ref, c_ref):
        xv = x_ref[...]
        a_ref[...] = jnp.dot(xv, wn_ref[...], preferred_element_type=jnp.float32, precision=lax.Precision.HIGHEST)
        c_ref[...] = jnp.dot(xv, wd_ref[...], preferred_element_type=jnp.float32, precision=lax.Precision.HIGHEST) + b_ref[...]

    return pl.pallas_call(
        body,
        grid=(R // BR,),
        in_specs=[
            pl.BlockSpec((BR, Cin), lambda i: (i, 0)),
            pl.BlockSpec((Cin, Ca), lambda i: (0, 0)),
            pl.BlockSpec((Cin, Cc), lambda i: (0, 0)),
            pl.BlockSpec((1, Cc), lambda i: (0, 0)),
        ],
        out_specs=[
            pl.BlockSpec((BR, Ca), lambda i: (i, 0)),
            pl.BlockSpec((BR, Cc), lambda i: (i, 0)),
        ],
        out_shape=[jax.ShapeDtypeStruct((R, Ca), jnp.float32),
                   jax.ShapeDtypeStruct((R, Cc), jnp.float32)],
    )(x, Wn, Wd, b2d)


def _tc_act_proj(m, c, Wn, Wd, b2d, BR):
    R, Cm = m.shape
    Cin = c.shape[1]
    Ca = Wn.shape[1]
    Cc = Wd.shape[1]

    def body(m_ref, c_ref, wn_ref, wd_ref, b_ref, x_ref, a_ref, cn_ref):
        t = m_ref[...][:, :Cin] + c_ref[...]
        xv = jnp.maximum(t, 0.2 * t)
        x_ref[...] = xv
        a_ref[...] = jnp.dot(xv, wn_ref[...], preferred_element_type=jnp.float32, precision=lax.Precision.HIGHEST)
        cn_ref[...] = jnp.dot(xv, wd_ref[...], preferred_element_type=jnp.float32, precision=lax.Precision.HIGHEST) + b_ref[...]

    return pl.pallas_call(
        body,
        grid=(R // BR,),
        in_specs=[
            pl.BlockSpec((BR, Cm), lambda i: (i, 0)),
            pl.BlockSpec((BR, Cin), lambda i: (i, 0)),
            pl.BlockSpec((Cin, Ca), lambda i: (0, 0)),
            pl.BlockSpec((Cin, Cc), lambda i: (0, 0)),
            pl.BlockSpec((1, Cc), lambda i: (0, 0)),
        ],
        out_specs=[
            pl.BlockSpec((BR, Cin), lambda i: (i, 0)),
            pl.BlockSpec((BR, Ca), lambda i: (i, 0)),
            pl.BlockSpec((BR, Cc), lambda i: (i, 0)),
        ],
        out_shape=[
            jax.ShapeDtypeStruct((R, Cin), jnp.float32),
            jax.ShapeDtypeStruct((R, Ca), jnp.float32),
            jax.ShapeDtypeStruct((R, Cc), jnp.float32),
        ],
    )(m, c, Wn, Wd, b2d)


def _tc_pool_proj(W, x0, x1, x2, m3, c3, Wn, Wd, b2d, NCH):
    B, M, N = W.shape
    Nc = N // NCH
    Gw = x0.shape[2] + x1.shape[2] + x2.shape[2] + m3.shape[2]
    Cout = Wn.shape[1]

    def body(w_ref, x0_ref, x1_ref, x2_ref, m3_ref, c3_ref, wn_ref, wd_ref,
             b_ref, a_ref, cn_ref, acc, wsum):
        nc = pl.program_id(1)
        w = w_ref[0]
        t = m3_ref[0] + c3_ref[0]
        x3 = jnp.maximum(t, 0.2 * t)
        g = jnp.concatenate([x0_ref[0], x1_ref[0], x2_ref[0], x3], axis=-1)
        pacc = jnp.dot(w, g, preferred_element_type=jnp.float32, precision=lax.Precision.HIGHEST)
        pw = jnp.sum(w, axis=1).reshape(M, 1)

        @pl.when(nc == 0)
        def _():
            acc[...] = pacc
            wsum[...] = pw

        @pl.when(nc > 0)
        def _():
            acc[...] += pacc
            wsum[...] += pw

        @pl.when(nc == NCH - 1)
        def _():
            vj = acc[...] / (wsum[...] + 1e-5)
            a_ref[0] = jnp.dot(vj, wn_ref[...], preferred_element_type=jnp.float32, precision=lax.Precision.HIGHEST)
            cn_ref[0] = jnp.dot(vj, wd_ref[...], preferred_element_type=jnp.float32, precision=lax.Precision.HIGHEST) + b_ref[...]

    return pl.pallas_call(
        body,
        grid=(B, NCH),
        in_specs=[
            pl.BlockSpec((1, M, Nc), lambda b, n: (b, 0, n)),
            pl.BlockSpec((1, Nc, x0.shape[2]), lambda b, n: (b, n, 0)),
            pl.BlockSpec((1, Nc, x1.shape[2]), lambda b, n: (b, n, 0)),
            pl.BlockSpec((1, Nc, x2.shape[2]), lambda b, n: (b, n, 0)),
            pl.BlockSpec((1, Nc, m3.shape[2]), lambda b, n: (b, n, 0)),
            pl.BlockSpec((1, Nc, c3.shape[2]), lambda b, n: (b, n, 0)),
            pl.BlockSpec((Gw, Cout), lambda b, n: (0, 0)),
            pl.BlockSpec((Gw, Cout), lambda b, n: (0, 0)),
            pl.BlockSpec((1, Cout), lambda b, n: (0, 0)),
        ],
        out_specs=[
            pl.BlockSpec((1, M, Cout), lambda b, n: (b, 0, 0)),
            pl.BlockSpec((1, M, Cout), lambda b, n: (b, 0, 0)),
        ],
        out_shape=[jax.ShapeDtypeStruct((B, M, Cout), jnp.float32)] * 2,
        scratch_shapes=[
            pltpu.VMEM((M, Gw), jnp.float32),
            pltpu.VMEM((M, 1), jnp.float32),
        ],
    )(W, x0, x1, x2, m3, c3, Wn, Wd, b2d)


def _tc_head(m3, c3, s1, s2, W1, b1, W2, b2, W3p, b3p):
    R = m3.shape[0]
    H1 = W1.shape[1]
    H2 = W2.shape[1]
    CO = W3p.shape[1]

    def body(m_ref, c_ref, s1_ref, s2_ref, w1_ref, b1_ref, w2_ref, b2_ref,
             w3_ref, b3_ref, o_ref):
        t = m_ref[...][:, :c_ref.shape[1]] + c_ref[...]
        s3 = jnp.maximum(t, 0.2 * t)
        j = jnp.concatenate([s1_ref[...], s2_ref[...], s3], axis=-1)
        h = jnp.dot(j, w1_ref[...], preferred_element_type=jnp.float32, precision=lax.Precision.HIGHEST) + b1_ref[...]
        h = jnp.maximum(h, 0.2 * h)
        h = jnp.dot(h, w2_ref[...], preferred_element_type=jnp.float32, precision=lax.Precision.HIGHEST) + b2_ref[...]
        h = jnp.maximum(h, 0.2 * h)
        o_ref[...] = jnp.dot(h, w3_ref[...], preferred_element_type=jnp.float32, precision=lax.Precision.HIGHEST) + b3_ref[...]

    return pl.pallas_call(
        body,
        out_shape=jax.ShapeDtypeStruct((R, CO), jnp.float32),
    )(m3, c3, s1, s2, W1, b1.reshape(1, H1), W2, b2.reshape(1, H2), W3p, b3p)


def _split_w(Wc, bc):
    Cin = Wc.shape[0] // 2
    return Wc[:Cin], Wc[Cin:] - Wc[:Cin], bc.reshape(1, -1)


def _flat_idx(idx, stride, R_pad):
    B, Rb, K = idx.shape
    idx = jnp.pad(idx.astype(jnp.int32), ((0, 0), (0, stride - Rb), (0, 0)))
    offs = (jnp.arange(B, dtype=jnp.int32) * stride)[:, None, None]
    return (idx + offs).reshape(R_pad * K)


def kernel(V, W, facesOneRingIdx, skeletonOneRingIdx,
           Wg1, bg1, Wg2, bg2, Wg3, bg3,
           Ws1, bs1, Ws2, bs2, Ws3, bs3,
           Wm1, bm1, Wm2, bm2, Wm3, bm3):
    B, N, _ = V.shape
    M = W.shape[1]
    K = facesOneRingIdx.shape[-1]

    NCH = 5
    N_pad = -(-N // (NCH * 128)) * (NCH * 128)
    R_pad = B * N_pad
    BR = 512
    RS = B * M

    Wn1, Wd1, b1 = _split_w(Wg1, bg1)
    Wn2, Wd2, b2 = _split_w(Wg2, bg2)
    Wn3, Wd3, b3 = _split_w(Wg3, bg3)
    Sn1, Sd1, sb1 = _split_w(Ws1, bs1)
    Sn2, Sd2, sb2 = _split_w(Ws2, bs2)
    Sn3, Sd3, sb3 = _split_w(Ws3, bs3)

    V_pad = jnp.pad(V, ((0, 0), (0, N_pad - N), (0, 0)))
    x0f = jnp.pad(V_pad.reshape(R_pad, 3), ((0, 0), (0, 5)))
    W_pad = jnp.pad(W, ((0, 0), (0, 0), (0, N_pad - N)))
    Wn1p = jnp.pad(Wn1, ((0, 5), (0, 64)))
    Wd1p = jnp.pad(Wd1, ((0, 5), (0, 0)))
    Sn3p = jnp.pad(Sn3, ((0, 0), (0, 64)))

    fidx = _flat_idx(facesOneRingIdx, N_pad, R_pad)
    sidx = _flat_idx(skeletonOneRingIdx, M, RS)

    a1, c1 = _tc_proj(x0f, Wn1p, Wd1p, b1, BR)
    m1 = _sc_gather_max(a1, fidx, 128, K, 4)
    x1, a2, c2 = _tc_act_proj(m1, c1, Wn2, Wd2, b2, BR)
    m2 = _sc_gather_max(a2, fidx, 128, K, 4)
    x2, a3, c3 = _tc_act_proj(m2, c2, Wn3, Wd3, b3, BR)
    m3 = _sc_gather_max(a3, fidx, 256, K, 4)

    def unflat(x, Cc):
        return x.reshape(B, N_pad, Cc)
    as1, cs1 = _tc_pool_proj(
        W_pad, V_pad, unflat(x1, 64), unflat(x2, 128), unflat(m3, 256),
        unflat(c3, 256), Sn1, Sd1, sb1, NCH=NCH)
    as1 = as1.reshape(RS, 256)
    cs1 = cs1.reshape(RS, 256)

    ms1 = _sc_gather_max(as1, sidx, 256, K, 4)
    s1, as2, cs2 = _tc_act_proj(ms1, cs1, Sn2, Sd2, sb2, RS)
    ms2 = _sc_gather_max(as2, sidx, 128, K, 4)
    s2, as3, cs3 = _tc_act_proj(ms2, cs2, Sn3p, Sd3, sb3, RS)
    ms3 = _sc_gather_max(as3, sidx, 128, K, 4)

    W3p = jnp.pad(Wm3, ((0, 0), (0, 125)))
    b3p = jnp.pad(bm3, (0, 125)).reshape(1, 128)
    out = _tc_head(ms3, cs3, s1, s2, Wm1, bm1, Wm2, bm2, W3p, b3p)
    return out[:, :3].reshape(B, M, 3)


# speedup vs baseline: 1.9023x; 1.7107x over previous
import functools

import jax
import jax.numpy as jnp
from jax import lax
from jax.experimental import pallas as pl
from jax.experimental.pallas import tpu as pltpu
from jax.experimental.pallas import tpu_sc as plsc

_NC = 2
_NS = 16
_NW = _NC * _NS
_LANES = 16


def _sc_gather_max(table, idx_flat, C, K, VPC):
    R = idx_flat.shape[0] // K
    RPW = R // _NW
    NCHUNK = RPW // VPC
    IPC = VPC * K
    NBUF = 4

    assert NCHUNK % NBUF == 0
    mesh = plsc.VectorSubcoreMesh(core_axis_name="c", subcore_axis_name="s")

    @functools.partial(
        pl.kernel,
        mesh=mesh,
        out_type=jax.ShapeDtypeStruct((R, C), jnp.float32),
        scratch_types=[
            pltpu.VMEM((RPW * K,), jnp.int32),
        ] + [pltpu.VMEM((IPC, C), jnp.float32)] * NBUF
          + [pltpu.VMEM((VPC, C), jnp.float32)] * NBUF
          + [pltpu.SemaphoreType.DMA] * NBUF
          + [pltpu.SemaphoreType.DMA] * NBUF,
    )
    def gk(table_hbm, idx_hbm, out_hbm, idx_v, *bufs):
        rows = bufs[:NBUF]
        outs = bufs[NBUF:2 * NBUF]
        gsem = bufs[2 * NBUF:3 * NBUF]
        osem = bufs[3 * NBUF:4 * NBUF]
        wid = lax.axis_index("s") * _NC + lax.axis_index("c")
        base = wid * RPW
        pltpu.sync_copy(idx_hbm.at[pl.ds(base * K, RPW * K)], idx_v)

        def fire(ci, b):
            pltpu.async_copy(
                table_hbm.at[idx_v.at[pl.ds(ci * IPC, IPC)]], rows[b], gsem[b])

        def out_slice(ci):
            return out_hbm.at[pl.ds(base + ci * VPC, VPC)]

        for b in range(NBUF):
            fire(b, b)

        def round_body(q, carry):
            for b in range(NBUF):
                ci = q * NBUF + b
                pltpu.make_async_copy(
                    table_hbm.at[idx_v.at[pl.ds(0, IPC)]], rows[b], gsem[b]
                ).wait()

                @pl.when(q > 0)
                def _(b=b, ci=ci):
                    pltpu.make_async_copy(
                        outs[b], out_slice(ci - NBUF), osem[b]).wait()
                for v in range(VPC):
                    def lane_body(c16, carry2, v=v, b=b):
                        off = c16 * _LANES
                        acc = rows[b][v * K, pl.ds(off, _LANES)]
                        for j in range(1, K):
                            acc = jnp.maximum(
                                acc, rows[b][v * K + j, pl.ds(off, _LANES)])
                        outs[b][v, pl.ds(off, _LANES)] = acc
                        return carry2
                    lax.fori_loop(0, C // _LANES, lane_body, 0)
                pltpu.async_copy(outs[b], out_slice(ci), osem[b])

                @pl.when(ci + NBUF < NCHUNK)
                def _(b=b, ci=ci):
                    fire(ci + NBUF, b)
            return carry
        lax.fori_loop(0, NCHUNK // NBUF, round_body, 0)

        for b in range(NBUF):
            pltpu.make_async_copy(
                outs[b], out_slice(NCHUNK - NBUF + b), osem[b]).wait()

    return gk(table, idx_flat)


def _sc_gather_max_spmem(table, idx_flat, C, K, VPC):
    R, _ = table.shape
    TPB = R // _NC
    RPW = TPB // _NS
    NCHUNK = RPW // VPC
    IPC = VPC * K
    NBUF = 4
    SLICE = TPB // _NS

    assert NCHUNK % NBUF == 0 and TPB % _NS == 0
    mesh = plsc.VectorSubcoreMesh(core_axis_name="c", subcore_axis_name="s")

    @functools.partial(
        pl.kernel,
        mesh=mesh,
        out_type=jax.ShapeDtypeStruct((R, C), jnp.float32),
        scratch_types=[
            pltpu.VMEM_SHARED((TPB, C), jnp.float32),
            pltpu.VMEM((RPW * K,), jnp.int32),
        ] + [pltpu.VMEM((IPC, C), jnp.float32)] * NBUF
          + [pltpu.VMEM((VPC, C), jnp.float32)] * NBUF
          + [pltpu.SemaphoreType.DMA] * NBUF
          + [pltpu.SemaphoreType.DMA] * NBUF,
    )
    def gk(table_hbm, idx_hbm, out_hbm, shared, idx_v, *bufs):
        rows = bufs[:NBUF]
        outs = bufs[NBUF:2 * NBUF]
        gsem = bufs[2 * NBUF:3 * NBUF]
        osem = bufs[3 * NBUF:4 * NBUF]
        sid = lax.axis_index("s")
        cid = lax.axis_index("c")
        pltpu.sync_copy(
            table_hbm.at[pl.ds(cid * TPB + sid * SLICE, SLICE)],
            shared.at[pl.ds(sid * SLICE, SLICE)])
        base = cid * TPB + sid * RPW
        pltpu.sync_copy(idx_hbm.at[pl.ds(base * K, RPW * K)], idx_v)
        plsc.subcore_barrier()

        def fire(ci, b):
            pltpu.async_copy(
                shared.at[idx_v.at[pl.ds(ci * IPC, IPC)]], rows[b], gsem[b])

        def out_slice(ci):
            return out_hbm.at[pl.ds(base + ci * VPC, VPC)]

        for b in range(NBUF):
            fire(b, b)

        def round_body(q, carry):
            for b in range(NBUF):
                ci = q * NBUF + b
                pltpu.make_async_copy(
                    shared.at[idx_v.at[pl.ds(0, IPC)]], rows[b], gsem[b]
                ).wait()

                @pl.when(q > 0)
                def _(b=b, ci=ci):
                    pltpu.make_async_copy(
                        outs[b], out_slice(ci - NBUF), osem[b]).wait()
                for v in range(VPC):
                    def lane_body(c16, carry2, v=v, b=b):
                        off = c16 * _LANES
                        acc = rows[b][v * K, pl.ds(off, _LANES)]
                        for j in range(1, K):
                            acc = jnp.maximum(
                                acc, rows[b][v * K + j, pl.ds(off, _LANES)])
                        outs[b][v, pl.ds(off, _LANES)] = acc
                        return carry2
                    lax.fori_loop(0, C // _LANES, lane_body, 0)
                pltpu.async_copy(outs[b], out_slice(ci), osem[b])

                @pl.when(ci + NBUF < NCHUNK)
                def _(b=b, ci=ci):
                    fire(ci + NBUF, b)
            return carry
        lax.fori_loop(0, NCHUNK // NBUF, round_body, 0)

        for b in range(NBUF):
            pltpu.make_async_copy(
                outs[b], out_slice(NCHUNK - NBUF + b), osem[b]).wait()

    return gk(table, idx_flat)


def _tc_proj(x, Wn, Wd, b2d, BR):
    R, Cin = x.shape
    Ca = Wn.shape[1]
    Cc = Wd.shape[1]

    def body(x_ref, wn_ref, wd_ref, b_ref, a_ref, c_ref):
        xv = x_ref[...]
        a_ref[...] = jnp.dot(xv, wn_ref[...], preferred_element_type=jnp.float32, precision=lax.Precision.HIGHEST)
        c_ref[...] = jnp.dot(xv, wd_ref[...], preferred_element_type=jnp.float32, precision=lax.Precision.HIGHEST) + b_ref[...]

    return pl.pallas_call(
        body,
        grid=(R // BR,),
        in_specs=[
            pl.BlockSpec((BR, Cin), lambda i: (i, 0)),
            pl.BlockSpec((Cin, Ca), lambda i: (0, 0)),
            pl.BlockSpec((Cin, Cc), lambda i: (0, 0)),
            pl.BlockSpec((1, Cc), lambda i: (0, 0)),
        ],
        out_specs=[
            pl.BlockSpec((BR, Ca), lambda i: (i, 0)),
            pl.BlockSpec((BR, Cc), lambda i: (i, 0)),
        ],
        out_shape=[jax.ShapeDtypeStruct((R, Ca), jnp.float32),
                   jax.ShapeDtypeStruct((R, Cc), jnp.float32)],
    )(x, Wn, Wd, b2d)


def _tc_act_proj(m, c, Wn, Wd, b2d, BR):
    R, Cm = m.shape
    Cin = c.shape[1]
    Ca = Wn.shape[1]
    Cc = Wd.shape[1]

    def body(m_ref, c_ref, wn_ref, wd_ref, b_ref, x_ref, a_ref, cn_ref):
        t = m_ref[...][:, :Cin] + c_ref[...]
        xv = jnp.maximum(t, 0.2 * t)
        x_ref[...] = xv
        a_ref[...] = jnp.dot(xv, wn_ref[...], preferred_element_type=jnp.float32, precision=lax.Precision.HIGHEST)
        cn_ref[...] = jnp.dot(xv, wd_ref[...], preferred_element_type=jnp.float32, precision=lax.Precision.HIGHEST) + b_ref[...]

    return pl.pallas_call(
        body,
        grid=(R // BR,),
        in_specs=[
            pl.BlockSpec((BR, Cm), lambda i: (i, 0)),
            pl.BlockSpec((BR, Cin), lambda i: (i, 0)),
            pl.BlockSpec((Cin, Ca), lambda i: (0, 0)),
            pl.BlockSpec((Cin, Cc), lambda i: (0, 0)),
            pl.BlockSpec((1, Cc), lambda i: (0, 0)),
        ],
        out_specs=[
            pl.BlockSpec((BR, Cin), lambda i: (i, 0)),
            pl.BlockSpec((BR, Ca), lambda i: (i, 0)),
            pl.BlockSpec((BR, Cc), lambda i: (i, 0)),
        ],
        out_shape=[
            jax.ShapeDtypeStruct((R, Cin), jnp.float32),
            jax.ShapeDtypeStruct((R, Ca), jnp.float32),
            jax.ShapeDtypeStruct((R, Cc), jnp.float32),
        ],
    )(m, c, Wn, Wd, b2d)


def _tc_pool_proj(W, x0, x1, x2, m3, c3, Wn, Wd, b2d, NCH):
    B, M, N = W.shape
    Nc = N // NCH
    Gw = x0.shape[2] + x1.shape[2] + x2.shape[2] + m3.shape[2]
    Cout = Wn.shape[1]

    def body(w_ref, x0_ref, x1_ref, x2_ref, m3_ref, c3_ref, wn_ref, wd_ref,
             b_ref, a_ref, cn_ref, acc, wsum):
        nc = pl.program_id(1)
        w = w_ref[0]
        t = m3_ref[0] + c3_ref[0]
        x3 = jnp.maximum(t, 0.2 * t)
        g = jnp.concatenate([x0_ref[0], x1_ref[0], x2_ref[0], x3], axis=-1)
        pacc = jnp.dot(w, g, preferred_element_type=jnp.float32, precision=lax.Precision.HIGHEST)
        pw = jnp.sum(w, axis=1).reshape(M, 1)

        @pl.when(nc == 0)
        def _():
            acc[...] = pacc
            wsum[...] = pw

        @pl.when(nc > 0)
        def _():
            acc[...] += pacc
            wsum[...] += pw

        @pl.when(nc == NCH - 1)
        def _():
            vj = acc[...] / (wsum[...] + 1e-5)
            a_ref[0] = jnp.dot(vj, wn_ref[...], preferred_element_type=jnp.float32, precision=lax.Precision.HIGHEST)
            cn_ref[0] = jnp.dot(vj, wd_ref[...], preferred_element_type=jnp.float32, precision=lax.Precision.HIGHEST) + b_ref[...]

    return pl.pallas_call(
        body,
        grid=(B, NCH),
        in_specs=[
            pl.BlockSpec((1, M, Nc), lambda b, n: (b, 0, n)),
            pl.BlockSpec((1, Nc, x0.shape[2]), lambda b, n: (b, n, 0)),
            pl.BlockSpec((1, Nc, x1.shape[2]), lambda b, n: (b, n, 0)),
            pl.BlockSpec((1, Nc, x2.shape[2]), lambda b, n: (b, n, 0)),
            pl.BlockSpec((1, Nc, m3.shape[2]), lambda b, n: (b, n, 0)),
            pl.BlockSpec((1, Nc, c3.shape[2]), lambda b, n: (b, n, 0)),
            pl.BlockSpec((Gw, Cout), lambda b, n: (0, 0)),
            pl.BlockSpec((Gw, Cout), lambda b, n: (0, 0)),
            pl.BlockSpec((1, Cout), lambda b, n: (0, 0)),
        ],
        out_specs=[
            pl.BlockSpec((1, M, Cout), lambda b, n: (b, 0, 0)),
            pl.BlockSpec((1, M, Cout), lambda b, n: (b, 0, 0)),
        ],
        out_shape=[jax.ShapeDtypeStruct((B, M, Cout), jnp.float32)] * 2,
        scratch_shapes=[
            pltpu.VMEM((M, Gw), jnp.float32),
            pltpu.VMEM((M, 1), jnp.float32),
        ],
    )(W, x0, x1, x2, m3, c3, Wn, Wd, b2d)


def _tc_head(m3, c3, s1, s2, W1, b1, W2, b2, W3p, b3p):
    R = m3.shape[0]
    H1 = W1.shape[1]
    H2 = W2.shape[1]
    CO = W3p.shape[1]

    def body(m_ref, c_ref, s1_ref, s2_ref, w1_ref, b1_ref, w2_ref, b2_ref,
             w3_ref, b3_ref, o_ref):
        t = m_ref[...][:, :c_ref.shape[1]] + c_ref[...]
        s3 = jnp.maximum(t, 0.2 * t)
        j = jnp.concatenate([s1_ref[...], s2_ref[...], s3], axis=-1)
        h = jnp.dot(j, w1_ref[...], preferred_element_type=jnp.float32, precision=lax.Precision.HIGHEST) + b1_ref[...]
        h = jnp.maximum(h, 0.2 * h)
        h = jnp.dot(h, w2_ref[...], preferred_element_type=jnp.float32, precision=lax.Precision.HIGHEST) + b2_ref[...]
        h = jnp.maximum(h, 0.2 * h)
        o_ref[...] = jnp.dot(h, w3_ref[...], preferred_element_type=jnp.float32, precision=lax.Precision.HIGHEST) + b3_ref[...]

    return pl.pallas_call(
        body,
        out_shape=jax.ShapeDtypeStruct((R, CO), jnp.float32),
    )(m3, c3, s1, s2, W1, b1.reshape(1, H1), W2, b2.reshape(1, H2), W3p, b3p)


def _split_w(Wc, bc):
    Cin = Wc.shape[0] // 2
    return Wc[:Cin], Wc[Cin:] - Wc[:Cin], bc.reshape(1, -1)


def _flat_idx(idx, stride, R_pad, local=False):
    B, Rb, K = idx.shape
    idx = jnp.pad(idx.astype(jnp.int32), ((0, 0), (0, stride - Rb), (0, 0)))
    if not local:
        offs = (jnp.arange(B, dtype=jnp.int32) * stride)[:, None, None]
        idx = idx + offs
    return idx.reshape(R_pad * K)


def kernel(V, W, facesOneRingIdx, skeletonOneRingIdx,
           Wg1, bg1, Wg2, bg2, Wg3, bg3,
           Ws1, bs1, Ws2, bs2, Ws3, bs3,
           Wm1, bm1, Wm2, bm2, Wm3, bm3):
    B, N, _ = V.shape
    M = W.shape[1]
    K = facesOneRingIdx.shape[-1]

    NCH = 5
    N_pad = -(-N // (NCH * 128)) * (NCH * 128)
    R_pad = B * N_pad
    BR = 512
    RS = B * M

    Wn1, Wd1, b1 = _split_w(Wg1, bg1)
    Wn2, Wd2, b2 = _split_w(Wg2, bg2)
    Wn3, Wd3, b3 = _split_w(Wg3, bg3)
    Sn1, Sd1, sb1 = _split_w(Ws1, bs1)
    Sn2, Sd2, sb2 = _split_w(Ws2, bs2)
    Sn3, Sd3, sb3 = _split_w(Ws3, bs3)

    V_pad = jnp.pad(V, ((0, 0), (0, N_pad - N), (0, 0)))
    x0f = jnp.pad(V_pad.reshape(R_pad, 3), ((0, 0), (0, 5)))
    W_pad = jnp.pad(W, ((0, 0), (0, 0), (0, N_pad - N)))
    Wn1p = jnp.pad(Wn1, ((0, 5), (0, 64)))
    Wd1p = jnp.pad(Wd1, ((0, 5), (0, 0)))
    Sn3p = jnp.pad(Sn3, ((0, 0), (0, 64)))

    fidx = _flat_idx(facesOneRingIdx, N_pad, R_pad)
    fidx_local = _flat_idx(facesOneRingIdx, N_pad, R_pad, local=True)
    sidx = _flat_idx(skeletonOneRingIdx, M, RS)

    a1, c1 = _tc_proj(x0f, Wn1p, Wd1p, b1, BR)
    m1 = _sc_gather_max_spmem(a1, fidx_local, 128, K, 4)
    x1, a2, c2 = _tc_act_proj(m1, c1, Wn2, Wd2, b2, BR)
    m2 = _sc_gather_max_spmem(a2, fidx_local, 128, K, 4)
    x2, a3, c3 = _tc_act_proj(m2, c2, Wn3, Wd3, b3, BR)
    m3 = _sc_gather_max(a3, fidx, 256, K, 4)

    def unflat(x, Cc):
        return x.reshape(B, N_pad, Cc)
    as1, cs1 = _tc_pool_proj(
        W_pad, V_pad, unflat(x1, 64), unflat(x2, 128), unflat(m3, 256),
        unflat(c3, 256), Sn1, Sd1, sb1, NCH=NCH)
    as1 = as1.reshape(RS, 256)
    cs1 = cs1.reshape(RS, 256)

    ms1 = _sc_gather_max(as1, sidx, 256, K, 4)
    s1, as2, cs2 = _tc_act_proj(ms1, cs1, Sn2, Sd2, sb2, RS)
    ms2 = _sc_gather_max(as2, sidx, 128, K, 4)
    s2, as3, cs3 = _tc_act_proj(ms2, cs2, Sn3p, Sd3, sb3, RS)
    ms3 = _sc_gather_max(as3, sidx, 128, K, 4)

    W3p = jnp.pad(Wm3, ((0, 0), (0, 125)))
    b3p = jnp.pad(bm3, (0, 125)).reshape(1, 128)
    out = _tc_head(ms3, cs3, s1, s2, Wm1, bm1, Wm2, bm2, W3p, b3p)
    return out[:, :3].reshape(B, M, 3)


# speedup vs baseline: 2.6233x; 1.3790x over previous
import functools

import jax
import jax.numpy as jnp
from jax import lax
from jax.experimental import pallas as pl
from jax.experimental.pallas import tpu as pltpu
from jax.experimental.pallas import tpu_sc as plsc

_NC = 2
_NS = 16
_NW = _NC * _NS
_LANES = 16


def _sc_gather_max(table, idx_flat, C, K, VPC):
    R = idx_flat.shape[0] // K
    RPW = R // _NW
    NCHUNK = RPW // VPC
    IPC = VPC * K
    NBUF = 4

    assert NCHUNK % NBUF == 0
    mesh = plsc.VectorSubcoreMesh(core_axis_name="c", subcore_axis_name="s")

    @functools.partial(
        pl.kernel,
        mesh=mesh,
        out_type=jax.ShapeDtypeStruct((R, C), jnp.float32),
        scratch_types=[
            pltpu.VMEM((RPW * K,), jnp.int32),
        ] + [pltpu.VMEM((IPC, C), jnp.float32)] * NBUF
          + [pltpu.VMEM((VPC, C), jnp.float32)] * NBUF
          + [pltpu.SemaphoreType.DMA] * NBUF
          + [pltpu.SemaphoreType.DMA] * NBUF,
    )
    def gk(table_hbm, idx_hbm, out_hbm, idx_v, *bufs):
        rows = bufs[:NBUF]
        outs = bufs[NBUF:2 * NBUF]
        gsem = bufs[2 * NBUF:3 * NBUF]
        osem = bufs[3 * NBUF:4 * NBUF]
        wid = lax.axis_index("s") * _NC + lax.axis_index("c")
        base = wid * RPW
        pltpu.sync_copy(idx_hbm.at[pl.ds(base * K, RPW * K)], idx_v)

        def fire(ci, b):
            pltpu.async_copy(
                table_hbm.at[idx_v.at[pl.ds(ci * IPC, IPC)]], rows[b], gsem[b])

        def out_slice(ci):
            return out_hbm.at[pl.ds(base + ci * VPC, VPC)]

        for b in range(NBUF):
            fire(b, b)

        def round_body(q, carry):
            for b in range(NBUF):
                ci = q * NBUF + b
                pltpu.make_async_copy(
                    table_hbm.at[idx_v.at[pl.ds(0, IPC)]], rows[b], gsem[b]
                ).wait()

                @pl.when(q > 0)
                def _(b=b, ci=ci):
                    pltpu.make_async_copy(
                        outs[b], out_slice(ci - NBUF), osem[b]).wait()
                for v in range(VPC):
                    def lane_body(c16, carry2, v=v, b=b):
                        off = c16 * _LANES
                        acc = rows[b][v * K, pl.ds(off, _LANES)]
                        for j in range(1, K):
                            acc = jnp.maximum(
                                acc, rows[b][v * K + j, pl.ds(off, _LANES)])
                        outs[b][v, pl.ds(off, _LANES)] = acc
                        return carry2
                    lax.fori_loop(0, C // _LANES, lane_body, 0)
                pltpu.async_copy(outs[b], out_slice(ci), osem[b])

                @pl.when(ci + NBUF < NCHUNK)
                def _(b=b, ci=ci):
                    fire(ci + NBUF, b)
            return carry
        lax.fori_loop(0, NCHUNK // NBUF, round_body, 0)

        for b in range(NBUF):
            pltpu.make_async_copy(
                outs[b], out_slice(NCHUNK - NBUF + b), osem[b]).wait()

    return gk(table, idx_flat)


def _sc_gather_max_spmem(table, idx_flat, C, K, VPC):
    R, _ = table.shape
    TPB = R // _NC
    RPW = TPB // _NS
    NCHUNK = RPW // VPC
    IPC = VPC * K
    NBUF = 4
    SLICE = TPB // _NS

    assert NCHUNK % NBUF == 0 and TPB % _NS == 0
    mesh = plsc.VectorSubcoreMesh(core_axis_name="c", subcore_axis_name="s")

    @functools.partial(
        pl.kernel,
        mesh=mesh,
        out_type=jax.ShapeDtypeStruct((R, C), jnp.float32),
        scratch_types=[
            pltpu.VMEM_SHARED((TPB, C), jnp.float32),
            pltpu.VMEM((RPW * K,), jnp.int32),
        ] + [pltpu.VMEM((IPC, C), jnp.float32)] * NBUF
          + [pltpu.VMEM((VPC, C), jnp.float32)] * NBUF
          + [pltpu.SemaphoreType.DMA] * NBUF
          + [pltpu.SemaphoreType.DMA] * NBUF,
    )
    def gk(table_hbm, idx_hbm, out_hbm, shared, idx_v, *bufs):
        rows = bufs[:NBUF]
        outs = bufs[NBUF:2 * NBUF]
        gsem = bufs[2 * NBUF:3 * NBUF]
        osem = bufs[3 * NBUF:4 * NBUF]
        sid = lax.axis_index("s")
        cid = lax.axis_index("c")
        pltpu.sync_copy(
            table_hbm.at[pl.ds(cid * TPB + sid * SLICE, SLICE)],
            shared.at[pl.ds(sid * SLICE, SLICE)])
        base = cid * TPB + sid * RPW
        pltpu.sync_copy(idx_hbm.at[pl.ds(base * K, RPW * K)], idx_v)
        plsc.subcore_barrier()

        def fire(ci, b):
            pltpu.async_copy(
                shared.at[idx_v.at[pl.ds(ci * IPC, IPC)]], rows[b], gsem[b])

        def out_slice(ci):
            return out_hbm.at[pl.ds(base + ci * VPC, VPC)]

        for b in range(NBUF):
            fire(b, b)

        def round_body(q, carry):
            for b in range(NBUF):
                ci = q * NBUF + b
                pltpu.make_async_copy(
                    shared.at[idx_v.at[pl.ds(0, IPC)]], rows[b], gsem[b]
                ).wait()

                @pl.when(q > 0)
                def _(b=b, ci=ci):
                    pltpu.make_async_copy(
                        outs[b], out_slice(ci - NBUF), osem[b]).wait()
                for v in range(VPC):
                    def lane_body(c16, carry2, v=v, b=b):
                        off = c16 * _LANES
                        acc = rows[b][v * K, pl.ds(off, _LANES)]
                        for j in range(1, K):
                            acc = jnp.maximum(
                                acc, rows[b][v * K + j, pl.ds(off, _LANES)])
                        outs[b][v, pl.ds(off, _LANES)] = acc
                        return carry2
                    lax.fori_loop(0, C // _LANES, lane_body, 0)
                pltpu.async_copy(outs[b], out_slice(ci), osem[b])

                @pl.when(ci + NBUF < NCHUNK)
                def _(b=b, ci=ci):
                    fire(ci + NBUF, b)
            return carry
        lax.fori_loop(0, NCHUNK // NBUF, round_body, 0)

        for b in range(NBUF):
            pltpu.make_async_copy(
                outs[b], out_slice(NCHUNK - NBUF + b), osem[b]).wait()

    return gk(table, idx_flat)


def _tc_proj(x, Wn, Wd, b2d, BR):
    R, Cin = x.shape
    Ca = Wn.shape[1]
    Cc = Wd.shape[1]

    def body(x_ref, wn_ref, wd_ref, b_ref, a_ref, c_ref):
        xv = x_ref[...]
        a_ref[...] = jnp.dot(xv, wn_ref[...], preferred_element_type=jnp.float32, precision=lax.Precision.HIGHEST)
        c_ref[...] = jnp.dot(xv, wd_ref[...], preferred_element_type=jnp.float32, precision=lax.Precision.HIGHEST) + b_ref[...]

    return pl.pallas_call(
        body,
        grid=(R // BR,),
        in_specs=[
            pl.BlockSpec((BR, Cin), lambda i: (i, 0)),
            pl.BlockSpec((Cin, Ca), lambda i: (0, 0)),
            pl.BlockSpec((Cin, Cc), lambda i: (0, 0)),
            pl.BlockSpec((1, Cc), lambda i: (0, 0)),
        ],
        out_specs=[
            pl.BlockSpec((BR, Ca), lambda i: (i, 0)),
            pl.BlockSpec((BR, Cc), lambda i: (i, 0)),
        ],
        out_shape=[jax.ShapeDtypeStruct((R, Ca), jnp.float32),
                   jax.ShapeDtypeStruct((R, Cc), jnp.float32)],
    )(x, Wn, Wd, b2d)


def _tc_act_proj(m, c, Wn, Wd, b2d, BR):
    R, Cm = m.shape
    Cin = c.shape[1]
    Ca = Wn.shape[1]
    Cc = Wd.shape[1]

    def body(m_ref, c_ref, wn_ref, wd_ref, b_ref, x_ref, a_ref, cn_ref):
        t = m_ref[...][:, :Cin] + c_ref[...]
        xv = jnp.maximum(t, 0.2 * t)
        x_ref[...] = xv
        a_ref[...] = jnp.dot(xv, wn_ref[...], preferred_element_type=jnp.float32, precision=lax.Precision.HIGHEST)
        cn_ref[...] = jnp.dot(xv, wd_ref[...], preferred_element_type=jnp.float32, precision=lax.Precision.HIGHEST) + b_ref[...]

    return pl.pallas_call(
        body,
        grid=(R // BR,),
        in_specs=[
            pl.BlockSpec((BR, Cm), lambda i: (i, 0)),
            pl.BlockSpec((BR, Cin), lambda i: (i, 0)),
            pl.BlockSpec((Cin, Ca), lambda i: (0, 0)),
            pl.BlockSpec((Cin, Cc), lambda i: (0, 0)),
            pl.BlockSpec((1, Cc), lambda i: (0, 0)),
        ],
        out_specs=[
            pl.BlockSpec((BR, Cin), lambda i: (i, 0)),
            pl.BlockSpec((BR, Ca), lambda i: (i, 0)),
            pl.BlockSpec((BR, Cc), lambda i: (i, 0)),
        ],
        out_shape=[
            jax.ShapeDtypeStruct((R, Cin), jnp.float32),
            jax.ShapeDtypeStruct((R, Ca), jnp.float32),
            jax.ShapeDtypeStruct((R, Cc), jnp.float32),
        ],
    )(m, c, Wn, Wd, b2d)


def _tc_pool_proj(W, x0, x1, x2, m3, c3, Wn, Wd, b2d, NCH):
    B, M, N = W.shape
    Nc = N // NCH
    Gw = x0.shape[2] + x1.shape[2] + x2.shape[2] + m3.shape[2]
    Cout = Wn.shape[1]

    def body(w_ref, x0_ref, x1_ref, x2_ref, m3_ref, c3_ref, wn_ref, wd_ref,
             b_ref, a_ref, cn_ref, acc, wsum):
        nc = pl.program_id(1)
        w = w_ref[0]
        t = m3_ref[0] + c3_ref[0]
        x3 = jnp.maximum(t, 0.2 * t)
        g = jnp.concatenate([x0_ref[0], x1_ref[0], x2_ref[0], x3], axis=-1)
        pacc = jnp.dot(w, g, preferred_element_type=jnp.float32, precision=lax.Precision.HIGHEST)
        pw = jnp.sum(w, axis=1).reshape(M, 1)

        @pl.when(nc == 0)
        def _():
            acc[...] = pacc
            wsum[...] = pw

        @pl.when(nc > 0)
        def _():
            acc[...] += pacc
            wsum[...] += pw

        @pl.when(nc == NCH - 1)
        def _():
            vj = acc[...] / (wsum[...] + 1e-5)
            a_ref[0] = jnp.dot(vj, wn_ref[...], preferred_element_type=jnp.float32, precision=lax.Precision.HIGHEST)
            cn_ref[0] = jnp.dot(vj, wd_ref[...], preferred_element_type=jnp.float32, precision=lax.Precision.HIGHEST) + b_ref[...]

    return pl.pallas_call(
        body,
        grid=(B, NCH),
        in_specs=[
            pl.BlockSpec((1, M, Nc), lambda b, n: (b, 0, n)),
            pl.BlockSpec((1, Nc, x0.shape[2]), lambda b, n: (b, n, 0)),
            pl.BlockSpec((1, Nc, x1.shape[2]), lambda b, n: (b, n, 0)),
            pl.BlockSpec((1, Nc, x2.shape[2]), lambda b, n: (b, n, 0)),
            pl.BlockSpec((1, Nc, m3.shape[2]), lambda b, n: (b, n, 0)),
            pl.BlockSpec((1, Nc, c3.shape[2]), lambda b, n: (b, n, 0)),
            pl.BlockSpec((Gw, Cout), lambda b, n: (0, 0)),
            pl.BlockSpec((Gw, Cout), lambda b, n: (0, 0)),
            pl.BlockSpec((1, Cout), lambda b, n: (0, 0)),
        ],
        out_specs=[
            pl.BlockSpec((1, M, Cout), lambda b, n: (b, 0, 0)),
            pl.BlockSpec((1, M, Cout), lambda b, n: (b, 0, 0)),
        ],
        out_shape=[jax.ShapeDtypeStruct((B, M, Cout), jnp.float32)] * 2,
        scratch_shapes=[
            pltpu.VMEM((M, Gw), jnp.float32),
            pltpu.VMEM((M, 1), jnp.float32),
        ],
    )(W, x0, x1, x2, m3, c3, Wn, Wd, b2d)


def _tc_head(m3, c3, s1, s2, W1, b1, W2, b2, W3p, b3p):
    R = m3.shape[0]
    H1 = W1.shape[1]
    H2 = W2.shape[1]
    CO = W3p.shape[1]

    def body(m_ref, c_ref, s1_ref, s2_ref, w1_ref, b1_ref, w2_ref, b2_ref,
             w3_ref, b3_ref, o_ref):
        t = m_ref[...][:, :c_ref.shape[1]] + c_ref[...]
        s3 = jnp.maximum(t, 0.2 * t)
        j = jnp.concatenate([s1_ref[...], s2_ref[...], s3], axis=-1)
        h = jnp.dot(j, w1_ref[...], preferred_element_type=jnp.float32, precision=lax.Precision.HIGHEST) + b1_ref[...]
        h = jnp.maximum(h, 0.2 * h)
        h = jnp.dot(h, w2_ref[...], preferred_element_type=jnp.float32, precision=lax.Precision.HIGHEST) + b2_ref[...]
        h = jnp.maximum(h, 0.2 * h)
        o_ref[...] = jnp.dot(h, w3_ref[...], preferred_element_type=jnp.float32, precision=lax.Precision.HIGHEST) + b3_ref[...]

    return pl.pallas_call(
        body,
        out_shape=jax.ShapeDtypeStruct((R, CO), jnp.float32),
    )(m3, c3, s1, s2, W1, b1.reshape(1, H1), W2, b2.reshape(1, H2), W3p, b3p)


def _split_w(Wc, bc):
    Cin = Wc.shape[0] // 2
    return Wc[:Cin], Wc[Cin:] - Wc[:Cin], bc.reshape(1, -1)


def _flat_idx(idx, stride, R_pad, local=False):
    B, Rb, K = idx.shape
    idx = jnp.pad(idx.astype(jnp.int32), ((0, 0), (0, stride - Rb), (0, 0)))
    if not local:
        offs = (jnp.arange(B, dtype=jnp.int32) * stride)[:, None, None]
        idx = idx + offs
    return idx.reshape(R_pad * K)


def kernel(V, W, facesOneRingIdx, skeletonOneRingIdx,
           Wg1, bg1, Wg2, bg2, Wg3, bg3,
           Ws1, bs1, Ws2, bs2, Ws3, bs3,
           Wm1, bm1, Wm2, bm2, Wm3, bm3):
    B, N, _ = V.shape
    M = W.shape[1]
    K = facesOneRingIdx.shape[-1]

    NCH = 5
    N_pad = -(-N // (NCH * 128)) * (NCH * 128)
    R_pad = B * N_pad
    BR = 512
    RS = B * M

    Wn1, Wd1, b1 = _split_w(Wg1, bg1)
    Wn2, Wd2, b2 = _split_w(Wg2, bg2)
    Wn3, Wd3, b3 = _split_w(Wg3, bg3)
    Sn1, Sd1, sb1 = _split_w(Ws1, bs1)
    Sn2, Sd2, sb2 = _split_w(Ws2, bs2)
    Sn3, Sd3, sb3 = _split_w(Ws3, bs3)

    V_pad = jnp.pad(V, ((0, 0), (0, N_pad - N), (0, 0)))
    x0f = jnp.pad(V_pad.reshape(R_pad, 3), ((0, 0), (0, 5)))
    W_pad = jnp.pad(W, ((0, 0), (0, 0), (0, N_pad - N)))
    Wn1p = jnp.pad(Wn1, ((0, 5), (0, 64)))
    Wd1p = jnp.pad(Wd1, ((0, 5), (0, 0)))
    Sn3p = jnp.pad(Sn3, ((0, 0), (0, 64)))

    fidx = _flat_idx(facesOneRingIdx, N_pad, R_pad)
    fidx_local = _flat_idx(facesOneRingIdx, N_pad, R_pad, local=True)
    sidx = _flat_idx(skeletonOneRingIdx, M, RS)

    a1, c1 = _tc_proj(x0f, Wn1p, Wd1p, b1, BR)
    m1 = _sc_gather_max_spmem(a1, fidx_local, 128, K, 4)
    x1, a2, c2 = _tc_act_proj(m1, c1, Wn2, Wd2, b2, BR)
    m2 = _sc_gather_max_spmem(a2, fidx_local, 128, K, 4)
    x2, a3, c3 = _tc_act_proj(m2, c2, Wn3, Wd3, b3, BR)
    m3a = _sc_gather_max_spmem(a3[:, :128], fidx_local, 128, K, 4)
    m3b = _sc_gather_max_spmem(a3[:, 128:], fidx_local, 128, K, 4)
    m3 = jnp.concatenate([m3a, m3b], axis=-1)

    def unflat(x, Cc):
        return x.reshape(B, N_pad, Cc)
    as1, cs1 = _tc_pool_proj(
        W_pad, V_pad, unflat(x1, 64), unflat(x2, 128), unflat(m3, 256),
        unflat(c3, 256), Sn1, Sd1, sb1, NCH=NCH)
    as1 = as1.reshape(RS, 256)
    cs1 = cs1.reshape(RS, 256)

    ms1 = _sc_gather_max(as1, sidx, 256, K, 4)
    s1, as2, cs2 = _tc_act_proj(ms1, cs1, Sn2, Sd2, sb2, RS)
    ms2 = _sc_gather_max(as2, sidx, 128, K, 4)
    s2, as3, cs3 = _tc_act_proj(ms2, cs2, Sn3p, Sd3, sb3, RS)
    ms3 = _sc_gather_max(as3, sidx, 128, K, 4)

    W3p = jnp.pad(Wm3, ((0, 0), (0, 125)))
    b3p = jnp.pad(bm3, (0, 125)).reshape(1, 128)
    out = _tc_head(ms3, cs3, s1, s2, Wm1, bm1, Wm2, bm2, W3p, b3p)
    return out[:, :3].reshape(B, M, 3)
